# speedup vs baseline: 1.1272x; 1.1272x over previous
import functools

import jax
import jax.numpy as jnp
from jax import lax
from jax.experimental import pallas as pl
from jax.experimental.pallas import tpu as pltpu
from jax.experimental.pallas import tpu_sc as plsc

NUM_CORES = 2
NUM_SUBCORES = 16
LANES = 16
NUM_WORKERS = NUM_CORES * NUM_SUBCORES


def _emb_kernel(B, S, D, SPW, ids_hbm, tok_hbm, pos_hbm, out_hbm,
                idx_v, pos_v, buf_v, sem):
    wid = lax.axis_index("s") * NUM_CORES + lax.axis_index("c")
    s0 = wid * SPW

    pltpu.sync_copy(pos_hbm.at[pl.ds(s0, SPW)], pos_v)
    for b in range(B):
        pltpu.sync_copy(ids_hbm.at[pl.ds(b * S + s0, SPW)], idx_v.at[b])

    for b in range(B):
        pltpu.async_copy(tok_hbm.at[idx_v.at[b]], buf_v, sem).wait()

        def row_body(i, carry):
            for c in range(D // LANES):
                sl = pl.ds(c * LANES, LANES)
                plsc.addupdate(buf_v.at[i, sl], pos_v[i, sl])
            return carry

        lax.fori_loop(0, SPW, row_body, 0)

        pltpu.sync_copy(buf_v, out_hbm.at[pl.ds(b * S + s0, SPW)])


def kernel(token_ids, tok_table, pos_table):
    B, S = token_ids.shape
    V, D = tok_table.shape
    assert S % NUM_WORKERS == 0 and D % LANES == 0
    SPW = S // NUM_WORKERS

    ids_flat = token_ids.reshape(-1).astype(jnp.int32)

    run = functools.partial(
        pl.kernel,
        mesh=plsc.VectorSubcoreMesh(core_axis_name="c", subcore_axis_name="s"),
        out_type=jax.ShapeDtypeStruct((B * S, D), jnp.float32),
        scratch_types=[
            pltpu.VMEM((B, SPW), jnp.int32),
            pltpu.VMEM((SPW, D), jnp.float32),
            pltpu.VMEM((SPW, D), jnp.float32),
            pltpu.SemaphoreType.DMA,
        ],
    )(functools.partial(_emb_kernel, B, S, D, SPW))

    out = run(ids_flat, tok_table, pos_table)
    return out.reshape(B, S, D)
